# baseline (speedup 1.0000x reference)
import functools

import jax
import jax.numpy as jnp
from jax import lax
from jax.experimental import pallas as pl
from jax.experimental.pallas import tpu as pltpu
from jax.experimental.pallas import tpu_sc as plsc

N = 10000
D = 128
E = 320000
N_ACC = 10112
ROWS_PER_TILE = N_ACC // 16
C = 128
NCHUNK_A = 160
NCHUNK_C = 80
CNTW = 128


def _sc_sums_body(nchunk, table, src_idx, dst_idx, zeros_f, sums,
                  is0, id0, is1, id1, is2, id2, is3, id3,
                  rows0, rows1, acc, sem0, sem1, sem2, sem3, g0, g1):
    c = lax.axis_index("c")
    s = lax.axis_index("s")
    r0 = s * ROWS_PER_TILE
    pltpu.sync_copy(zeros_f.at[pl.ds(r0, ROWS_PER_TILE)],
                    acc.at[pl.ds(r0, ROWS_PER_TILE)])
    plsc.subcore_barrier()

    ib = ((is0, id0, sem0), (is1, id1, sem1), (is2, id2, sem2),
          (is3, id3, sem3))
    rb = ((rows0, g0), (rows1, g1))

    def idx_start(k, u):
        bs, bd, sem = ib[u]
        pltpu.async_copy(src_idx.at[c, s, k], bs, sem)
        pltpu.async_copy(dst_idx.at[c, s, k], bd, sem)

    def idx_wait(k, u):
        bs, bd, sem = ib[u]
        pltpu.make_async_copy(src_idx.at[c, s, k], bs, sem).wait()
        pltpu.make_async_copy(dst_idx.at[c, s, k], bd, sem).wait()

    def gather_start(u, p):
        pltpu.async_copy(table.at[ib[u][0]], rb[p][0], rb[p][1])

    def gather_wait(u, p):
        pltpu.make_async_copy(table.at[ib[u][0]], rb[p][0], rb[p][1]).wait()

    def scatter(u, p):
        pltpu.sync_copy(rb[p][0], acc.at[ib[u][1]], add=True)

    for k in range(4):
        idx_start(k, k)
    idx_wait(0, 0)
    gather_start(0, 0)

    def quad(q, carry):
        k0 = q * 4

        def step(u, p):
            nxt = (u + 1) % 4
            gather_wait(u, p)
            if u < 3:
                idx_wait(k0 + u + 1, nxt)
                gather_start(nxt, 1 - p)
            else:
                @pl.when(k0 + 4 < nchunk)
                def _():
                    idx_wait(k0 + 4, 0)
                    gather_start(0, 1 - p)
            scatter(u, p)

            @pl.when(k0 + u + 4 < nchunk)
            def _():
                idx_start(k0 + u + 4, u)

        step(0, 0)
        step(1, 1)
        step(2, 0)
        step(3, 1)
        return carry

    lax.fori_loop(0, nchunk // 4, quad, 0)
    plsc.subcore_barrier()
    pltpu.sync_copy(acc.at[pl.ds(r0, ROWS_PER_TILE)],
                    sums.at[c, pl.ds(r0, ROWS_PER_TILE)])


def _sc_counts_body(dst_idx, ones_blk, zeros_c, cnts,
                    id0, id1, id2, id3, ones_v, cacc,
                    sem0, sem1, sem2, sem3):
    c = lax.axis_index("c")
    s = lax.axis_index("s")
    r0 = s * ROWS_PER_TILE
    pltpu.sync_copy(ones_blk, ones_v)
    pltpu.sync_copy(zeros_c.at[pl.ds(r0, ROWS_PER_TILE)],
                    cacc.at[pl.ds(r0, ROWS_PER_TILE)])
    plsc.subcore_barrier()

    ib = ((id0, sem0), (id1, sem1), (id2, sem2), (id3, sem3))

    def idx_start(k, u):
        pltpu.async_copy(dst_idx.at[c, s, k], ib[u][0], ib[u][1])

    def idx_wait(k, u):
        pltpu.make_async_copy(dst_idx.at[c, s, k], ib[u][0], ib[u][1]).wait()

    for k in range(4):
        idx_start(k, k)

    def quad(q, carry):
        k0 = q * 4
        for u in range(4):
            idx_wait(k0 + u, u)
            pltpu.sync_copy(ones_v, cacc.at[ib[u][0]], add=True)

            @pl.when(k0 + u + 4 < NCHUNK_A)
            def _():
                idx_start(k0 + u + 4, u)
        return carry

    lax.fori_loop(0, NCHUNK_A // 4, quad, 0)
    plsc.subcore_barrier()
    pltpu.sync_copy(cacc.at[pl.ds(r0, ROWS_PER_TILE)],
                    cnts.at[c, pl.ds(r0, ROWS_PER_TILE)])


@functools.lru_cache(maxsize=None)
def _sc_stages():
    mesh = plsc.VectorSubcoreMesh(core_axis_name="c", subcore_axis_name="s",
                                  num_cores=2, num_subcores=16)

    def sums_kernel(nchunk):
        return pl.kernel(
            functools.partial(_sc_sums_body, nchunk),
            out_type=jax.ShapeDtypeStruct((2, N_ACC, D), jnp.float32),
            mesh=mesh,
            scratch_types=(
                [pltpu.VMEM((C,), jnp.int32)] * 8 +
                [pltpu.VMEM((C, D), jnp.float32)] * 2 +
                [pltpu.VMEM_SHARED((N_ACC, D), jnp.float32)] +
                [pltpu.SemaphoreType.DMA] * 6
            ),
        )

    counts_kernel = pl.kernel(
        _sc_counts_body,
        out_type=jax.ShapeDtypeStruct((2, N_ACC, CNTW), jnp.float32),
        mesh=mesh,
        scratch_types=(
            [pltpu.VMEM((C,), jnp.int32)] * 4 +
            [pltpu.VMEM((C, CNTW), jnp.float32)] +
            [pltpu.VMEM_SHARED((N_ACC, CNTW), jnp.float32)] +
            [pltpu.SemaphoreType.DMA] * 4
        ),
    )
    return sums_kernel(NCHUNK_A), sums_kernel(NCHUNK_C), counts_kernel


_BLK = 1000


def _tc_update_body(s_ref, c_ref, x_ref, w_ref, b_ref, o_ref):
    inv = 1.0 / jnp.maximum(c_ref[0, :, 0:1], 1.0)
    agg = s_ref[0] * inv
    xcat = jnp.concatenate([agg, x_ref[...]], axis=1)
    o_ref[...] = jnp.maximum(
        jnp.dot(xcat, w_ref[...], preferred_element_type=jnp.float32)
        + b_ref[...], 0.0)


def _tc_update(sums, cnts, rel, x, w, b):
    return pl.pallas_call(
        _tc_update_body,
        out_shape=jax.ShapeDtypeStruct((N, D), jnp.float32),
        grid=(N // _BLK,),
        in_specs=[
            pl.BlockSpec((1, _BLK, D), lambda i: (rel, i, 0)),
            pl.BlockSpec((1, _BLK, CNTW), lambda i: (rel, i, 0)),
            pl.BlockSpec((_BLK, D), lambda i: (i, 0)),
            pl.BlockSpec((2 * D, D), lambda i: (0, 0)),
            pl.BlockSpec((1, D), lambda i: (0, 0)),
        ],
        out_specs=pl.BlockSpec((_BLK, D), lambda i: (i, 0)),
    )(sums, cnts, x, w, b.reshape(1, D))


def _tc_update2_body(s_ref, c_ref, x_ref, w_ref, b_ref, o_ref):
    inv = 1.0 / jnp.maximum(c_ref[0, :, 0:1], 1.0)
    agg = (s_ref[0] + s_ref[1]) * inv
    xcat = jnp.concatenate([agg, x_ref[...]], axis=1)
    o_ref[...] = jnp.maximum(
        jnp.dot(xcat, w_ref[...], preferred_element_type=jnp.float32)
        + b_ref[...], 0.0)


def _tc_update2(sums2, cnts, rel, x, w, b):
    return pl.pallas_call(
        _tc_update2_body,
        out_shape=jax.ShapeDtypeStruct((N, D), jnp.float32),
        grid=(N // _BLK,),
        in_specs=[
            pl.BlockSpec((2, _BLK, D), lambda i: (0, i, 0)),
            pl.BlockSpec((1, _BLK, CNTW), lambda i: (rel, i, 0)),
            pl.BlockSpec((_BLK, D), lambda i: (i, 0)),
            pl.BlockSpec((2 * D, D), lambda i: (0, 0)),
            pl.BlockSpec((1, D), lambda i: (0, 0)),
        ],
        out_specs=pl.BlockSpec((_BLK, D), lambda i: (i, 0)),
    )(sums2, cnts, x, w, b.reshape(1, D))


def _pack_idx(flat, lead, per_tile, nchunk, pad_val):
    a = flat.reshape(lead, 16, per_tile)
    a = jnp.pad(a, ((0, 0), (0, 0), (0, nchunk * C - per_tile)),
                constant_values=pad_val)
    return a.reshape(lead, 16, nchunk, C)


def kernel(x_node, x_link, edge_index_nl, edge_index_ln,
           W_nl_0, b_nl_0, W_ln_0, b_ln_0,
           W_nl_1, b_nl_1, W_ln_1, b_ln_1):
    src_nl, dst_nl = edge_index_nl[0], edge_index_nl[1]
    src_ln, dst_ln = edge_index_ln[0], edge_index_ln[1]

    table0 = jnp.concatenate([x_node, x_link], axis=0)
    srcA = _pack_idx(jnp.concatenate([src_nl, src_ln + N]), 2, E // 16,
                     NCHUNK_A, 0)
    dstA = _pack_idx(jnp.concatenate([dst_nl, dst_ln]), 2, E // 16,
                     NCHUNK_A, N)
    srcC = _pack_idx(src_nl, 2, E // 32, NCHUNK_C, 0)
    dstC = _pack_idx(dst_nl, 2, E // 32, NCHUNK_C, N)

    ones_blk = jnp.ones((C, CNTW), jnp.float32)
    zeros_f = jnp.zeros((N_ACC, D), jnp.float32)
    zeros_c = jnp.zeros((N_ACC, CNTW), jnp.float32)

    sums_a, sums_c, counts = _sc_stages()
    cnts = counts(dstA, ones_blk, zeros_c)
    sums1 = sums_a(table0, srcA, dstA, zeros_f)
    xl1 = _tc_update(sums1, cnts, 0, x_link, W_nl_0, b_nl_0)
    xn1 = _tc_update(sums1, cnts, 1, x_node, W_ln_0, b_ln_0)
    sums2 = sums_c(xn1, srcC, dstC, zeros_f)
    xl2 = _tc_update2(sums2, cnts, 0, xl1, W_nl_1, b_nl_1)
    return xl2


# speedup vs baseline: 1.0607x; 1.0607x over previous
import functools

import jax
import jax.numpy as jnp
from jax import lax
from jax.experimental import pallas as pl
from jax.experimental.pallas import tpu as pltpu
from jax.experimental.pallas import tpu_sc as plsc

N = 10000
D = 128
E = 320000
N_ACC = 10112
ROWS_PER_TILE = N_ACC // 16
C = 64
NCHUNK_A = 320
NCHUNK_C = 160
CNTW = 128


def _sc_sums_body(nchunk, table, src_idx, dst_idx, zeros_f, sums,
                  is0, id0, is1, id1, is2, id2, is3, id3,
                  is4, id4, is5, id5, is6, id6, is7, id7,
                  rows0, rows1, rows2, rows3, acc,
                  sem0, sem1, sem2, sem3, sem4, sem5, sem6, sem7,
                  g0, g1, g2, g3):
    c = lax.axis_index("c")
    s = lax.axis_index("s")
    r0 = s * ROWS_PER_TILE
    pltpu.sync_copy(zeros_f.at[pl.ds(r0, ROWS_PER_TILE)],
                    acc.at[pl.ds(r0, ROWS_PER_TILE)])
    plsc.subcore_barrier()

    ib = ((is0, id0, sem0), (is1, id1, sem1), (is2, id2, sem2),
          (is3, id3, sem3), (is4, id4, sem4), (is5, id5, sem5),
          (is6, id6, sem6), (is7, id7, sem7))
    rb = ((rows0, g0), (rows1, g1), (rows2, g2), (rows3, g3))

    def idx_start(k, v):
        bs, bd, sem = ib[v]
        pltpu.async_copy(src_idx.at[c, s, k], bs, sem)
        pltpu.async_copy(dst_idx.at[c, s, k], bd, sem)

    def idx_wait(k, v):
        bs, bd, sem = ib[v]
        pltpu.make_async_copy(src_idx.at[c, s, k], bs, sem).wait()
        pltpu.make_async_copy(dst_idx.at[c, s, k], bd, sem).wait()

    def gather_start(v, p):
        pltpu.async_copy(table.at[ib[v][0]], rb[p][0], rb[p][1])

    def gather_wait(v, p):
        pltpu.make_async_copy(table.at[ib[v][0]], rb[p][0], rb[p][1]).wait()

    def scatter(v, p):
        pltpu.sync_copy(rb[p][0], acc.at[ib[v][1]], add=True)

    for k in range(8):
        idx_start(k, k)
    for k in range(4):
        idx_wait(k, k)
        gather_start(k, k)

    def octet(q, carry):
        k0 = q * 8
        for u in range(8):
            k = k0 + u
            v, p = u, u % 4
            gather_wait(v, p)
            scatter(v, p)

            @pl.when(k + 8 < nchunk)
            def _():
                idx_start(k + 8, v)

            @pl.when(k + 4 < nchunk)
            def _():
                idx_wait(k + 4, (u + 4) % 8)
                gather_start((u + 4) % 8, p)
        return carry

    lax.fori_loop(0, nchunk // 8, octet, 0)
    plsc.subcore_barrier()
    pltpu.sync_copy(acc.at[pl.ds(r0, ROWS_PER_TILE)],
                    sums.at[c, pl.ds(r0, ROWS_PER_TILE)])


def _sc_counts_body(dst_idx, ones_blk, zeros_c, cnts,
                    id0, id1, id2, id3, ones_v, cacc,
                    sem0, sem1, sem2, sem3):
    c = lax.axis_index("c")
    s = lax.axis_index("s")
    r0 = s * ROWS_PER_TILE
    pltpu.sync_copy(ones_blk, ones_v)
    pltpu.sync_copy(zeros_c.at[pl.ds(r0, ROWS_PER_TILE)],
                    cacc.at[pl.ds(r0, ROWS_PER_TILE)])
    plsc.subcore_barrier()

    ib = ((id0, sem0), (id1, sem1), (id2, sem2), (id3, sem3))

    def idx_start(k, u):
        pltpu.async_copy(dst_idx.at[c, s, k], ib[u][0], ib[u][1])

    def idx_wait(k, u):
        pltpu.make_async_copy(dst_idx.at[c, s, k], ib[u][0], ib[u][1]).wait()

    for k in range(4):
        idx_start(k, k)

    def quad(q, carry):
        k0 = q * 4
        for u in range(4):
            idx_wait(k0 + u, u)
            pltpu.sync_copy(ones_v, cacc.at[ib[u][0]], add=True)

            @pl.when(k0 + u + 4 < NCHUNK_A)
            def _():
                idx_start(k0 + u + 4, u)
        return carry

    lax.fori_loop(0, NCHUNK_A // 4, quad, 0)
    plsc.subcore_barrier()
    pltpu.sync_copy(cacc.at[pl.ds(r0, ROWS_PER_TILE)],
                    cnts.at[c, pl.ds(r0, ROWS_PER_TILE)])


@functools.lru_cache(maxsize=None)
def _sc_stages():
    mesh = plsc.VectorSubcoreMesh(core_axis_name="c", subcore_axis_name="s",
                                  num_cores=2, num_subcores=16)

    def sums_kernel(nchunk):
        return pl.kernel(
            functools.partial(_sc_sums_body, nchunk),
            out_type=jax.ShapeDtypeStruct((2, N_ACC, D), jnp.float32),
            mesh=mesh,
            scratch_types=(
                [pltpu.VMEM((C,), jnp.int32)] * 16 +
                [pltpu.VMEM((C, D), jnp.float32)] * 4 +
                [pltpu.VMEM_SHARED((N_ACC, D), jnp.float32)] +
                [pltpu.SemaphoreType.DMA] * 12
            ),
        )

    counts_kernel = pl.kernel(
        _sc_counts_body,
        out_type=jax.ShapeDtypeStruct((2, N_ACC, CNTW), jnp.float32),
        mesh=mesh,
        scratch_types=(
            [pltpu.VMEM((C,), jnp.int32)] * 4 +
            [pltpu.VMEM((C, CNTW), jnp.float32)] +
            [pltpu.VMEM_SHARED((N_ACC, CNTW), jnp.float32)] +
            [pltpu.SemaphoreType.DMA] * 4
        ),
    )
    return sums_kernel(NCHUNK_A), sums_kernel(NCHUNK_C), counts_kernel


_BLK = 1000


def _tc_update_body(s_ref, c_ref, x_ref, w_ref, b_ref, o_ref):
    inv = 1.0 / jnp.maximum(c_ref[0, :, 0:1], 1.0)
    agg = s_ref[0] * inv
    xcat = jnp.concatenate([agg, x_ref[...]], axis=1)
    o_ref[...] = jnp.maximum(
        jnp.dot(xcat, w_ref[...], preferred_element_type=jnp.float32)
        + b_ref[...], 0.0)


def _tc_update(sums, cnts, rel, x, w, b):
    return pl.pallas_call(
        _tc_update_body,
        out_shape=jax.ShapeDtypeStruct((N, D), jnp.float32),
        grid=(N // _BLK,),
        in_specs=[
            pl.BlockSpec((1, _BLK, D), lambda i: (rel, i, 0)),
            pl.BlockSpec((1, _BLK, CNTW), lambda i: (rel, i, 0)),
            pl.BlockSpec((_BLK, D), lambda i: (i, 0)),
            pl.BlockSpec((2 * D, D), lambda i: (0, 0)),
            pl.BlockSpec((1, D), lambda i: (0, 0)),
        ],
        out_specs=pl.BlockSpec((_BLK, D), lambda i: (i, 0)),
    )(sums, cnts, x, w, b.reshape(1, D))


def _tc_update2_body(s_ref, c_ref, x_ref, w_ref, b_ref, o_ref):
    inv = 1.0 / jnp.maximum(c_ref[0, :, 0:1], 1.0)
    agg = (s_ref[0] + s_ref[1]) * inv
    xcat = jnp.concatenate([agg, x_ref[...]], axis=1)
    o_ref[...] = jnp.maximum(
        jnp.dot(xcat, w_ref[...], preferred_element_type=jnp.float32)
        + b_ref[...], 0.0)


def _tc_update2(sums2, cnts, rel, x, w, b):
    return pl.pallas_call(
        _tc_update2_body,
        out_shape=jax.ShapeDtypeStruct((N, D), jnp.float32),
        grid=(N // _BLK,),
        in_specs=[
            pl.BlockSpec((2, _BLK, D), lambda i: (0, i, 0)),
            pl.BlockSpec((1, _BLK, CNTW), lambda i: (rel, i, 0)),
            pl.BlockSpec((_BLK, D), lambda i: (i, 0)),
            pl.BlockSpec((2 * D, D), lambda i: (0, 0)),
            pl.BlockSpec((1, D), lambda i: (0, 0)),
        ],
        out_specs=pl.BlockSpec((_BLK, D), lambda i: (i, 0)),
    )(sums2, cnts, x, w, b.reshape(1, D))


def _pack_idx(flat, lead, per_tile, nchunk, pad_val):
    a = flat.reshape(lead, 16, per_tile)
    a = jnp.pad(a, ((0, 0), (0, 0), (0, nchunk * C - per_tile)),
                constant_values=pad_val)
    return a.reshape(lead, 16, nchunk, C)


def kernel(x_node, x_link, edge_index_nl, edge_index_ln,
           W_nl_0, b_nl_0, W_ln_0, b_ln_0,
           W_nl_1, b_nl_1, W_ln_1, b_ln_1):
    src_nl, dst_nl = edge_index_nl[0], edge_index_nl[1]
    src_ln, dst_ln = edge_index_ln[0], edge_index_ln[1]

    table0 = jnp.concatenate([x_node, x_link], axis=0)
    srcA = _pack_idx(jnp.concatenate([src_nl, src_ln + N]), 2, E // 16,
                     NCHUNK_A, 0)
    dstA = _pack_idx(jnp.concatenate([dst_nl, dst_ln]), 2, E // 16,
                     NCHUNK_A, N)
    srcC = _pack_idx(src_nl, 2, E // 32, NCHUNK_C, 0)
    dstC = _pack_idx(dst_nl, 2, E // 32, NCHUNK_C, N)

    ones_blk = jnp.ones((C, CNTW), jnp.float32)
    zeros_f = jnp.zeros((N_ACC, D), jnp.float32)
    zeros_c = jnp.zeros((N_ACC, CNTW), jnp.float32)

    sums_a, sums_c, counts = _sc_stages()
    cnts = counts(dstA, ones_blk, zeros_c)
    sums1 = sums_a(table0, srcA, dstA, zeros_f)
    xl1 = _tc_update(sums1, cnts, 0, x_link, W_nl_0, b_nl_0)
    xn1 = _tc_update(sums1, cnts, 1, x_node, W_ln_0, b_ln_0)
    sums2 = sums_c(xn1, srcC, dstC, zeros_f)
    xl2 = _tc_update2(sums2, cnts, 0, xl1, W_nl_1, b_nl_1)
    return xl2
